# baseline (speedup 1.0000x reference)
import functools

import jax
import jax.numpy as jnp
from jax import lax
from jax.experimental import pallas as pl
from jax.experimental.pallas import tpu as pltpu
from jax.experimental.pallas import tpu_sc as plsc

N = 10000
E = 320000
D = 128
H = 64
R = 8
NPAD = 10240
CHUNK = 128
CHUNKS = 160
EPAD = 16 * CHUNKS * CHUNK
ROWS_PER_TILE = NPAD // 16
ZCHUNK = 128
NZ = ROWS_PER_TILE // ZCHUNK
NBLK = 50
BLK = 200


def _mm_body(x_ref, wlo_ref, whi_ref, wself_ref, tlo_ref, thi_ref, tself_ref):
    x = x_ref[...]
    tlo_ref[...] = jnp.dot(x, wlo_ref[...], preferred_element_type=jnp.float32)
    thi_ref[...] = jnp.dot(x, whi_ref[...], preferred_element_type=jnp.float32)
    tself_ref[...] = jnp.dot(x, wself_ref[...], preferred_element_type=jnp.float32)


def _idx_body(src_ref, typ_ref, out_ref):
    out_ref[...] = src_ref[...] * R + typ_ref[...]


def _combine_body(tself_ref, b_ref, agg_ref, deg_ref, out_ref):
    deg = jnp.maximum(deg_ref[0, :, 0:1] + deg_ref[1, :, 0:1], 1.0)
    agg = jnp.concatenate([agg_ref[0], agg_ref[1]], axis=1)
    out_ref[...] = jnp.maximum(
        tself_ref[...] + b_ref[0:1, :] + agg / deg, 0.0)


NBUF = 4


def _sc_deg_kernel(didx, z16, ones16, deg_out,
                   didx_v, ones_v, deg_sh):
    c = lax.axis_index("c")
    s = lax.axis_index("s")
    half = CHUNKS // 2

    pltpu.sync_copy(ones16, ones_v)
    for k in range(NZ):
        base = s * ROWS_PER_TILE + k * ZCHUNK
        pltpu.sync_copy(z16, deg_sh.at[pl.ds(base, ZCHUNK)])
    plsc.subcore_barrier()

    pltpu.sync_copy(didx.at[pl.ds(s * CHUNKS + c * half, half)], didx_v)

    def body(j, carry):
        pltpu.sync_copy(ones_v, deg_sh.at[didx_v.at[j]], add=True)
        return carry

    lax.fori_loop(0, half, body, 0)
    plsc.subcore_barrier()

    pltpu.sync_copy(deg_sh.at[pl.ds(s * ROWS_PER_TILE, ROWS_PER_TILE)],
                    deg_out.at[pl.ds(c * NPAD + s * ROWS_PER_TILE,
                                     ROWS_PER_TILE)])


def _sc_edge_kernel(t9_lo, t9_hi, gidx, didx, z64,
                    agg_out,
                    gidx_v, didx_v, rows0_v, rows1_v, rows2_v, rows3_v,
                    agg_sh,
                    g0, g1, g2, g3, s0, s1, s2, s3):
    c = lax.axis_index("c")
    s = lax.axis_index("s")
    rows = [rows0_v, rows1_v, rows2_v, rows3_v]
    gsems = [g0, g1, g2, g3]
    ssems = [s0, s1, s2, s3]

    for k in range(NZ):
        base = s * ROWS_PER_TILE + k * ZCHUNK
        pltpu.sync_copy(z64, agg_sh.at[pl.ds(base, ZCHUNK)])
    plsc.subcore_barrier()

    pltpu.sync_copy(gidx.at[pl.ds(s * CHUNKS, CHUNKS)], gidx_v)
    pltpu.sync_copy(didx.at[pl.ds(s * CHUNKS, CHUNKS)], didx_v)

    def start_gather(j, b):
        @pl.when(c == 0)
        def _():
            pltpu.async_copy(t9_lo.at[gidx_v.at[j]], rows[b], gsems[b])

        @pl.when(c == 1)
        def _():
            pltpu.async_copy(t9_hi.at[gidx_v.at[j]], rows[b], gsems[b])

    def wait_gather(j, b):
        @pl.when(c == 0)
        def _():
            pltpu.make_async_copy(
                t9_lo.at[gidx_v.at[j]], rows[b], gsems[b]).wait()

        @pl.when(c == 1)
        def _():
            pltpu.make_async_copy(
                t9_hi.at[gidx_v.at[j]], rows[b], gsems[b]).wait()

    def start_scatter(j, b):
        pltpu.async_copy(rows[b], agg_sh.at[didx_v.at[j]], ssems[b], add=True)

    def wait_scatter(j, b):
        pltpu.make_async_copy(
            rows[b], agg_sh.at[didx_v.at[j]], ssems[b]).wait()

    for b in range(NBUF):
        start_gather(b, b)

    def body(i, carry):
        for b in range(NBUF):
            j = i * NBUF + b
            wait_gather(j, b)
            jm = j - (NBUF - 1)
            bn = (b + 1) % NBUF

            @pl.when(jm >= 0)
            def _(jm=jm, bn=bn, j=j):
                wait_scatter(jm, bn)

                @pl.when(j + 1 < CHUNKS)
                def _():
                    start_gather(j + 1, bn)

            start_scatter(j, b)

        return carry

    lax.fori_loop(0, CHUNKS // NBUF, body, 0)

    for k in range(NBUF - 1):
        j = CHUNKS - (NBUF - 1) + k
        wait_scatter(j, j % NBUF)
    plsc.subcore_barrier()

    pltpu.sync_copy(agg_sh.at[pl.ds(s * ROWS_PER_TILE, ROWS_PER_TILE)],
                    agg_out.at[pl.ds(c * NPAD + s * ROWS_PER_TILE,
                                     ROWS_PER_TILE)])


def kernel(x, edge_index, edge_type, rel_weight, W_self, b_self):
    f32 = jnp.float32
    x = x.astype(f32)
    src = edge_index[0].astype(jnp.int32)
    dst = edge_index[1].astype(jnp.int32)
    typ = edge_type.astype(jnp.int32)

    rw = jnp.transpose(rel_weight.astype(f32), (1, 0, 2))
    w_lo = rw[:, :, :H].reshape(D, R * H)
    w_hi = rw[:, :, H:].reshape(D, R * H)

    t_lo, t_hi, t_self = pl.pallas_call(
        _mm_body,
        grid=(NBLK,),
        in_specs=[pl.BlockSpec((BLK, D), lambda i: (i, 0)),
                  pl.BlockSpec((D, R * H), lambda i: (0, 0)),
                  pl.BlockSpec((D, R * H), lambda i: (0, 0)),
                  pl.BlockSpec((D, D), lambda i: (0, 0))],
        out_specs=[pl.BlockSpec((BLK, R * H), lambda i: (i, 0)),
                   pl.BlockSpec((BLK, R * H), lambda i: (i, 0)),
                   pl.BlockSpec((BLK, D), lambda i: (i, 0))],
        out_shape=[jax.ShapeDtypeStruct((N, R * H), f32),
                   jax.ShapeDtypeStruct((N, R * H), f32),
                   jax.ShapeDtypeStruct((N, D), f32)],
    )(x, w_lo, w_hi, W_self.astype(f32))
    t9_lo = t_lo.reshape(N * R, H)
    t9_hi = t_hi.reshape(N * R, H)

    pad = EPAD - E
    srcp = jnp.concatenate([src, jnp.zeros((pad,), jnp.int32)]).reshape(-1, CHUNK)
    typp = jnp.concatenate([typ, jnp.zeros((pad,), jnp.int32)]).reshape(-1, CHUNK)
    didx = jnp.concatenate([dst, jnp.full((pad,), N, jnp.int32)]).reshape(-1, CHUNK)

    gidx = pl.pallas_call(
        _idx_body,
        in_specs=[pl.BlockSpec(srcp.shape, lambda: (0, 0))] * 2,
        out_specs=pl.BlockSpec(srcp.shape, lambda: (0, 0)),
        out_shape=jax.ShapeDtypeStruct(srcp.shape, jnp.int32),
    )(srcp, typp)

    z64 = jnp.zeros((ZCHUNK, H), f32)
    z16 = jnp.zeros((ZCHUNK, 16), f32)
    ones16 = jnp.ones((CHUNK, 16), f32)

    mesh = plsc.VectorSubcoreMesh(core_axis_name="c", subcore_axis_name="s")
    deg_call = functools.partial(
        pl.kernel, mesh=mesh,
        out_type=jax.ShapeDtypeStruct((2 * NPAD, 16), f32),
        compiler_params=pltpu.CompilerParams(use_tc_tiling_on_sc=False),
        scratch_types=[
            pltpu.VMEM((CHUNKS // 2, CHUNK), jnp.int32),
            pltpu.VMEM((CHUNK, 16), f32),
            pltpu.VMEM_SHARED((NPAD, 16), f32),
        ],
    )(_sc_deg_kernel)
    deg_flat = deg_call(didx, z16, ones16)
    degtab = deg_flat.reshape(2, NPAD, 16)

    sc_call = functools.partial(
        pl.kernel, mesh=mesh,
        out_type=jax.ShapeDtypeStruct((2 * NPAD, H), f32),
        compiler_params=pltpu.CompilerParams(use_tc_tiling_on_sc=False),
        scratch_types=[
            pltpu.VMEM((CHUNKS, CHUNK), jnp.int32),
            pltpu.VMEM((CHUNKS, CHUNK), jnp.int32),
            pltpu.VMEM((CHUNK, H), f32),
            pltpu.VMEM((CHUNK, H), f32),
            pltpu.VMEM((CHUNK, H), f32),
            pltpu.VMEM((CHUNK, H), f32),
            pltpu.VMEM_SHARED((NPAD, H), f32),
        ] + [pltpu.SemaphoreType.DMA] * 8,
    )(_sc_edge_kernel)
    agg_flat = sc_call(t9_lo, t9_hi, gidx, didx, z64)
    agg = agg_flat.reshape(2, NPAD, H)

    b_tile = jnp.tile(b_self.astype(f32).reshape(1, D), (8, 1))
    out = pl.pallas_call(
        _combine_body,
        grid=(NBLK,),
        in_specs=[pl.BlockSpec((BLK, D), lambda i: (i, 0)),
                  pl.BlockSpec((8, D), lambda i: (0, 0)),
                  pl.BlockSpec((2, BLK, H), lambda i: (0, i, 0)),
                  pl.BlockSpec((2, BLK, 16), lambda i: (0, i, 0))],
        out_specs=pl.BlockSpec((BLK, D), lambda i: (i, 0)),
        out_shape=jax.ShapeDtypeStruct((N, D), f32),
    )(t_self, b_tile, agg, degtab)
    return out


# speedup vs baseline: 1.0301x; 1.0301x over previous
import functools

import jax
import jax.numpy as jnp
from jax import lax
from jax.experimental import pallas as pl
from jax.experimental.pallas import tpu as pltpu
from jax.experimental.pallas import tpu_sc as plsc

N = 10000
E = 320000
D = 128
H = 64
R = 8
NPAD = 10240
CHUNK = 128
CHUNKS = 160
EPAD = 16 * CHUNKS * CHUNK
ROWS_PER_TILE = NPAD // 16
ZCHUNK = 128
NZ = ROWS_PER_TILE // ZCHUNK
NBLK = 50
BLK = 200


def _mm_body(x_ref, wlo_ref, whi_ref, wself_ref, tlo_ref, thi_ref, tself_ref):
    x = x_ref[...]
    tlo_ref[...] = jnp.dot(x, wlo_ref[...], preferred_element_type=jnp.float32)
    thi_ref[...] = jnp.dot(x, whi_ref[...], preferred_element_type=jnp.float32)
    tself_ref[...] = jnp.dot(x, wself_ref[...], preferred_element_type=jnp.float32)


def _idx_body(src_ref, typ_ref, out_ref):
    out_ref[...] = src_ref[...] * R + typ_ref[...]


def _combine_body(tself_ref, b_ref, agg_ref, deg_ref, out_ref):
    deg = jnp.maximum(deg_ref[0, :, 0:1] + deg_ref[1, :, 0:1], 1.0)
    agg = jnp.concatenate([agg_ref[0], agg_ref[1]], axis=1)
    out_ref[...] = jnp.maximum(
        tself_ref[...] + b_ref[0:1, :] + agg / deg, 0.0)


def _sc_edge_kernel(t9_lo, t9_hi, gidx, didx, z64, z16, ones16,
                    agg_out, deg_out,
                    gidx_v, didx_v, rows0_v, rows1_v, z64_v, z16_v, ones_v,
                    agg_sh, deg_sh, sem0, sem1):
    c = lax.axis_index("c")
    s = lax.axis_index("s")

    pltpu.sync_copy(z64, z64_v)
    pltpu.sync_copy(z16, z16_v)
    pltpu.sync_copy(ones16, ones_v)

    for k in range(NZ):
        base = s * ROWS_PER_TILE + k * ZCHUNK
        pltpu.sync_copy(z64_v, agg_sh.at[pl.ds(base, ZCHUNK)])
        pltpu.sync_copy(z16_v, deg_sh.at[pl.ds(base, ZCHUNK)])
    plsc.subcore_barrier()

    pltpu.sync_copy(gidx.at[pl.ds(s * CHUNKS, CHUNKS)], gidx_v)
    pltpu.sync_copy(didx.at[pl.ds(s * CHUNKS, CHUNKS)], didx_v)

    def start_gather(j, rows, sem):
        @pl.when(c == 0)
        def _():
            pltpu.async_copy(t9_lo.at[gidx_v.at[j]], rows, sem)

        @pl.when(c == 1)
        def _():
            pltpu.async_copy(t9_hi.at[gidx_v.at[j]], rows, sem)

    def wait_gather(j, rows, sem):
        @pl.when(c == 0)
        def _():
            pltpu.make_async_copy(t9_lo.at[gidx_v.at[j]], rows, sem).wait()

        @pl.when(c == 1)
        def _():
            pltpu.make_async_copy(t9_hi.at[gidx_v.at[j]], rows, sem).wait()

    def scatter(j, rows):
        pltpu.sync_copy(rows, agg_sh.at[didx_v.at[j]], add=True)
        @pl.when((j < CHUNKS // 2) == (c == 0))
        def _():
            pltpu.sync_copy(ones_v, deg_sh.at[didx_v.at[j]], add=True)

    start_gather(0, rows0_v, sem0)

    def body(i, carry):
        j0 = 2 * i
        wait_gather(j0, rows0_v, sem0)
        start_gather(j0 + 1, rows1_v, sem1)
        scatter(j0, rows0_v)
        wait_gather(j0 + 1, rows1_v, sem1)

        @pl.when(i < CHUNKS // 2 - 1)
        def _():
            start_gather(j0 + 2, rows0_v, sem0)

        scatter(j0 + 1, rows1_v)
        return carry

    lax.fori_loop(0, CHUNKS // 2, body, 0)
    plsc.subcore_barrier()

    pltpu.sync_copy(agg_sh.at[pl.ds(s * ROWS_PER_TILE, ROWS_PER_TILE)],
                    agg_out.at[pl.ds(c * NPAD + s * ROWS_PER_TILE,
                                     ROWS_PER_TILE)])
    pltpu.sync_copy(deg_sh.at[pl.ds(s * ROWS_PER_TILE, ROWS_PER_TILE)],
                    deg_out.at[pl.ds(c * NPAD + s * ROWS_PER_TILE,
                                     ROWS_PER_TILE)])


def kernel(x, edge_index, edge_type, rel_weight, W_self, b_self):
    f32 = jnp.float32
    x = x.astype(f32)
    src = edge_index[0].astype(jnp.int32)
    dst = edge_index[1].astype(jnp.int32)
    typ = edge_type.astype(jnp.int32)

    rw = jnp.transpose(rel_weight.astype(f32), (1, 0, 2))
    w_lo = rw[:, :, :H].reshape(D, R * H)
    w_hi = rw[:, :, H:].reshape(D, R * H)

    t_lo, t_hi, t_self = pl.pallas_call(
        _mm_body,
        grid=(NBLK,),
        in_specs=[pl.BlockSpec((BLK, D), lambda i: (i, 0)),
                  pl.BlockSpec((D, R * H), lambda i: (0, 0)),
                  pl.BlockSpec((D, R * H), lambda i: (0, 0)),
                  pl.BlockSpec((D, D), lambda i: (0, 0))],
        out_specs=[pl.BlockSpec((BLK, R * H), lambda i: (i, 0)),
                   pl.BlockSpec((BLK, R * H), lambda i: (i, 0)),
                   pl.BlockSpec((BLK, D), lambda i: (i, 0))],
        out_shape=[jax.ShapeDtypeStruct((N, R * H), f32),
                   jax.ShapeDtypeStruct((N, R * H), f32),
                   jax.ShapeDtypeStruct((N, D), f32)],
    )(x, w_lo, w_hi, W_self.astype(f32))
    t9_lo = t_lo.reshape(N * R, H)
    t9_hi = t_hi.reshape(N * R, H)

    pad = EPAD - E
    srcp = jnp.concatenate([src, jnp.zeros((pad,), jnp.int32)]).reshape(-1, CHUNK)
    typp = jnp.concatenate([typ, jnp.zeros((pad,), jnp.int32)]).reshape(-1, CHUNK)
    didx = jnp.concatenate([dst, jnp.full((pad,), N, jnp.int32)]).reshape(-1, CHUNK)

    gidx = pl.pallas_call(
        _idx_body,
        in_specs=[pl.BlockSpec(srcp.shape, lambda: (0, 0))] * 2,
        out_specs=pl.BlockSpec(srcp.shape, lambda: (0, 0)),
        out_shape=jax.ShapeDtypeStruct(srcp.shape, jnp.int32),
    )(srcp, typp)

    z64 = jnp.zeros((ZCHUNK, H), f32)
    z16 = jnp.zeros((ZCHUNK, 16), f32)
    ones16 = jnp.ones((CHUNK, 16), f32)

    mesh = plsc.VectorSubcoreMesh(core_axis_name="c", subcore_axis_name="s")
    sc_call = functools.partial(
        pl.kernel, mesh=mesh,
        out_type=(jax.ShapeDtypeStruct((2 * NPAD, H), f32),
                  jax.ShapeDtypeStruct((2 * NPAD, 16), f32)),
        compiler_params=pltpu.CompilerParams(use_tc_tiling_on_sc=False),
        scratch_types=[
            pltpu.VMEM((CHUNKS, CHUNK), jnp.int32),
            pltpu.VMEM((CHUNKS, CHUNK), jnp.int32),
            pltpu.VMEM((CHUNK, H), f32),
            pltpu.VMEM((CHUNK, H), f32),
            pltpu.VMEM((ZCHUNK, H), f32),
            pltpu.VMEM((ZCHUNK, 16), f32),
            pltpu.VMEM((CHUNK, 16), f32),
            pltpu.VMEM_SHARED((NPAD, H), f32),
            pltpu.VMEM_SHARED((NPAD, 16), f32),
            pltpu.SemaphoreType.DMA,
            pltpu.SemaphoreType.DMA,
        ],
    )(_sc_edge_kernel)
    agg_flat, deg_flat = sc_call(t9_lo, t9_hi, gidx, didx, z64, z16, ones16)
    agg = agg_flat.reshape(2, NPAD, H)
    degtab = deg_flat.reshape(2, NPAD, 16)

    b_tile = jnp.tile(b_self.astype(f32).reshape(1, D), (8, 1))
    out = pl.pallas_call(
        _combine_body,
        grid=(NBLK,),
        in_specs=[pl.BlockSpec((BLK, D), lambda i: (i, 0)),
                  pl.BlockSpec((8, D), lambda i: (0, 0)),
                  pl.BlockSpec((2, BLK, H), lambda i: (0, i, 0)),
                  pl.BlockSpec((2, BLK, 16), lambda i: (0, i, 0))],
        out_specs=pl.BlockSpec((BLK, D), lambda i: (i, 0)),
        out_shape=jax.ShapeDtypeStruct((N, D), f32),
    )(t_self, b_tile, agg, degtab)
    return out


# speedup vs baseline: 1.6094x; 1.5623x over previous
import functools

import jax
import jax.numpy as jnp
from jax import lax
from jax.experimental import pallas as pl
from jax.experimental.pallas import tpu as pltpu
from jax.experimental.pallas import tpu_sc as plsc

N = 10000
E = 320000
D = 128
H = 64
R = 8
NPAD = 10240
CHUNK = 128
ECH = E // CHUNK
BIGT = 4
CHT = 157
HALF = 78
ROWS_PER_TILE = NPAD // 16
ZCHUNK = 128
NZ = ROWS_PER_TILE // ZCHUNK
NBLK = 50
BLK = 200


def _mm_body(x_ref, wlo_ref, whi_ref, wself_ref, tlo_ref, thi_ref, tself_ref):
    x = x_ref[...]
    tlo_ref[...] = jnp.dot(x, wlo_ref[...], preferred_element_type=jnp.float32)
    thi_ref[...] = jnp.dot(x, whi_ref[...], preferred_element_type=jnp.float32)
    tself_ref[...] = jnp.dot(x, wself_ref[...], preferred_element_type=jnp.float32)


def _idx_body(src_ref, typ_ref, out_ref):
    out_ref[...] = src_ref[...] * R + typ_ref[...]


def _combine_body(tself_ref, b_ref, agg_ref, deg_ref, out_ref):
    deg = jnp.maximum(deg_ref[0, :, 0:1] + deg_ref[1, :, 0:1], 1.0)
    agg = jnp.concatenate([agg_ref[0], agg_ref[1]], axis=1)
    out_ref[...] = jnp.maximum(
        tself_ref[...] + b_ref[0:1, :] + agg / deg, 0.0)


def _sc_edge_kernel(t9_lo, t9_hi, gidx, didx, z64, z16, ones16, dummy_g,
                    dummy_d,
                    agg_out, deg_out,
                    gidx_v, didx_v, rows0_v, rows1_v, z64_v, z16_v, ones_v,
                    agg_sh, deg_sh, sem0, sem1):
    c = lax.axis_index("c")
    s = lax.axis_index("s")

    pltpu.sync_copy(z64, z64_v)
    pltpu.sync_copy(z16, z16_v)
    pltpu.sync_copy(ones16, ones_v)

    for k in range(NZ):
        base = s * ROWS_PER_TILE + k * ZCHUNK
        pltpu.sync_copy(z64_v, agg_sh.at[pl.ds(base, ZCHUNK)])
        pltpu.sync_copy(z16_v, deg_sh.at[pl.ds(base, ZCHUNK)])
    plsc.subcore_barrier()

    @pl.when(s < BIGT)
    def _():
        pltpu.sync_copy(gidx.at[pl.ds(s * CHT, CHT)], gidx_v)
        pltpu.sync_copy(didx.at[pl.ds(s * CHT, CHT)], didx_v)

    @pl.when(s >= BIGT)
    def _():
        start = s * (CHT - 1) + BIGT
        pltpu.sync_copy(gidx.at[pl.ds(start, CHT - 1)],
                        gidx_v.at[pl.ds(0, CHT - 1)])
        pltpu.sync_copy(didx.at[pl.ds(start, CHT - 1)],
                        didx_v.at[pl.ds(0, CHT - 1)])
        pltpu.sync_copy(dummy_g, gidx_v.at[pl.ds(CHT - 1, 1)])
        pltpu.sync_copy(dummy_d, didx_v.at[pl.ds(CHT - 1, 1)])

    def start_gather(j, rows, sem):
        @pl.when(c == 0)
        def _():
            pltpu.async_copy(t9_lo.at[gidx_v.at[j]], rows, sem)

        @pl.when(c == 1)
        def _():
            pltpu.async_copy(t9_hi.at[gidx_v.at[j]], rows, sem)

    def wait_gather(j, rows, sem):
        @pl.when(c == 0)
        def _():
            pltpu.make_async_copy(t9_lo.at[gidx_v.at[j]], rows, sem).wait()

        @pl.when(c == 1)
        def _():
            pltpu.make_async_copy(t9_hi.at[gidx_v.at[j]], rows, sem).wait()

    def scatter(j, rows):
        pltpu.sync_copy(rows, agg_sh.at[didx_v.at[j]], add=True)
        @pl.when((j < HALF) == (c == 0))
        def _():
            pltpu.sync_copy(ones_v, deg_sh.at[didx_v.at[j]], add=True)

    start_gather(0, rows0_v, sem0)

    def body(i, carry):
        j0 = 2 * i
        wait_gather(j0, rows0_v, sem0)
        start_gather(j0 + 1, rows1_v, sem1)
        scatter(j0, rows0_v)
        wait_gather(j0 + 1, rows1_v, sem1)
        start_gather(j0 + 2, rows0_v, sem0)
        scatter(j0 + 1, rows1_v)
        return carry

    lax.fori_loop(0, CHT // 2, body, 0)
    wait_gather(CHT - 1, rows0_v, sem0)
    scatter(CHT - 1, rows0_v)
    plsc.subcore_barrier()

    pltpu.sync_copy(agg_sh.at[pl.ds(s * ROWS_PER_TILE, ROWS_PER_TILE)],
                    agg_out.at[pl.ds(c * NPAD + s * ROWS_PER_TILE,
                                     ROWS_PER_TILE)])
    pltpu.sync_copy(deg_sh.at[pl.ds(s * ROWS_PER_TILE, ROWS_PER_TILE)],
                    deg_out.at[pl.ds(c * NPAD + s * ROWS_PER_TILE,
                                     ROWS_PER_TILE)])


def kernel(x, edge_index, edge_type, rel_weight, W_self, b_self):
    f32 = jnp.float32
    x = x.astype(f32)
    src = edge_index[0].astype(jnp.int32)
    dst = edge_index[1].astype(jnp.int32)
    typ = edge_type.astype(jnp.int32)

    rw = jnp.transpose(rel_weight.astype(f32), (1, 0, 2))
    w_lo = rw[:, :, :H].reshape(D, R * H)
    w_hi = rw[:, :, H:].reshape(D, R * H)

    t_lo, t_hi, t_self = pl.pallas_call(
        _mm_body,
        grid=(NBLK,),
        in_specs=[pl.BlockSpec((BLK, D), lambda i: (i, 0)),
                  pl.BlockSpec((D, R * H), lambda i: (0, 0)),
                  pl.BlockSpec((D, R * H), lambda i: (0, 0)),
                  pl.BlockSpec((D, D), lambda i: (0, 0))],
        out_specs=[pl.BlockSpec((BLK, R * H), lambda i: (i, 0)),
                   pl.BlockSpec((BLK, R * H), lambda i: (i, 0)),
                   pl.BlockSpec((BLK, D), lambda i: (i, 0))],
        out_shape=[jax.ShapeDtypeStruct((N, R * H), f32),
                   jax.ShapeDtypeStruct((N, R * H), f32),
                   jax.ShapeDtypeStruct((N, D), f32)],
    )(x, w_lo, w_hi, W_self.astype(f32))
    t9_lo = t_lo.reshape(N * R, H)
    t9_hi = t_hi.reshape(N * R, H)

    srcp = src.reshape(ECH, CHUNK)
    typp = typ.reshape(ECH, CHUNK)
    didx = dst.reshape(ECH, CHUNK)

    gidx = pl.pallas_call(
        _idx_body,
        in_specs=[pl.BlockSpec(srcp.shape, lambda: (0, 0))] * 2,
        out_specs=pl.BlockSpec(srcp.shape, lambda: (0, 0)),
        out_shape=jax.ShapeDtypeStruct(srcp.shape, jnp.int32),
    )(srcp, typp)

    z64 = jnp.zeros((ZCHUNK, H), f32)
    z16 = jnp.zeros((ZCHUNK, 16), f32)
    ones16 = jnp.ones((CHUNK, 16), f32)
    dummy_g = jnp.zeros((1, CHUNK), jnp.int32)
    dummy_d = jnp.full((1, CHUNK), N, jnp.int32)

    mesh = plsc.VectorSubcoreMesh(core_axis_name="c", subcore_axis_name="s")
    sc_call = functools.partial(
        pl.kernel, mesh=mesh,
        out_type=(jax.ShapeDtypeStruct((2 * NPAD, H), f32),
                  jax.ShapeDtypeStruct((2 * NPAD, 16), f32)),
        compiler_params=pltpu.CompilerParams(use_tc_tiling_on_sc=False),
        scratch_types=[
            pltpu.VMEM((CHT, CHUNK), jnp.int32),
            pltpu.VMEM((CHT, CHUNK), jnp.int32),
            pltpu.VMEM((CHUNK, H), f32),
            pltpu.VMEM((CHUNK, H), f32),
            pltpu.VMEM((ZCHUNK, H), f32),
            pltpu.VMEM((ZCHUNK, 16), f32),
            pltpu.VMEM((CHUNK, 16), f32),
            pltpu.VMEM_SHARED((NPAD, H), f32),
            pltpu.VMEM_SHARED((NPAD, 16), f32),
            pltpu.SemaphoreType.DMA,
            pltpu.SemaphoreType.DMA,
        ],
    )(_sc_edge_kernel)
    agg_flat, deg_flat = sc_call(t9_lo, t9_hi, gidx, didx, z64, z16, ones16,
                                 dummy_g, dummy_d)
    agg = agg_flat.reshape(2, NPAD, H)
    degtab = deg_flat.reshape(2, NPAD, 16)

    b_tile = jnp.tile(b_self.astype(f32).reshape(1, D), (8, 1))
    out = pl.pallas_call(
        _combine_body,
        grid=(NBLK,),
        in_specs=[pl.BlockSpec((BLK, D), lambda i: (i, 0)),
                  pl.BlockSpec((8, D), lambda i: (0, 0)),
                  pl.BlockSpec((2, BLK, H), lambda i: (0, i, 0)),
                  pl.BlockSpec((2, BLK, 16), lambda i: (0, i, 0))],
        out_specs=pl.BlockSpec((BLK, D), lambda i: (i, 0)),
        out_shape=jax.ShapeDtypeStruct((N, D), f32),
    )(t_self, b_tile, agg, degtab)
    return out
